# baseline (device time: 27520 ns/iter reference)
import jax
import jax.numpy as jnp
from jax import lax
from jax.experimental import pallas as pl
from jax.experimental.pallas import tpu as pltpu


def kernel(O, Wo):
    B, S_full, H_loc, D = O.shape
    K = H_loc * D
    N = Wo.shape[1]
    S_half = S_full // 2
    NH = N // 2
    NC = 128
    CPB = NH // NC
    C = B * CPB
    LAG = 5

    O2 = O.reshape(B, S_full, K)

    def body(o_ref, w_ref, out_ref, o_bf, w_bf, ysend, yrecv, xrecv,
             ysend_sems, yrecv_sems, fsend_sems, xrecv_sems):
        my_x = lax.axis_index("x")
        my_y = lax.axis_index("y")
        ypeer = (my_x, 1 - my_y)
        xpeer = (1 - my_x, my_y)

        barrier = pltpu.get_barrier_semaphore()
        for p in (ypeer, xpeer):
            pl.semaphore_signal(barrier, inc=1, device_id=p,
                                device_id_type=pl.DeviceIdType.MESH)
        pl.semaphore_wait(barrier, 2)

        peer_s0 = (1 - my_y) * S_half
        my_s0 = my_y * S_half
        nbase = my_x * NH
        obase = (1 - my_x) * NH

        w_bf[:, pl.ds(nbase, NH)] = w_ref[:, pl.ds(nbase, NH)].astype(
            jnp.bfloat16)

        y_rdmas = []
        for b in range(B):
            o_bf[b, pl.ds(peer_s0, S_half), :] = o_ref[
                b, pl.ds(peer_s0, S_half), :].astype(jnp.bfloat16)
            for q in range(CPB):
                c = b * CPB + q
                ysend[c] = jnp.dot(
                    o_bf[b, pl.ds(peer_s0, S_half), :],
                    w_bf[:, pl.ds(nbase + q * NC, NC)],
                    preferred_element_type=jnp.float32).astype(jnp.bfloat16)
                r = pltpu.make_async_remote_copy(
                    src_ref=ysend.at[c], dst_ref=yrecv.at[c],
                    send_sem=ysend_sems.at[c], recv_sem=yrecv_sems.at[c],
                    device_id=ypeer, device_id_type=pl.DeviceIdType.MESH)
                r.start()
                y_rdmas.append(r)

        w_bf[:, pl.ds(obase, NH)] = w_ref[:, pl.ds(obase, NH)].astype(
            jnp.bfloat16)

        def add_x_chunk(c):
            bb, qq = c // CPB, c % CPB
            f_rdmas[c].wait_recv()
            out_ref[bb, :, pl.ds(obase + qq * NC, NC)] = (
                out_ref[bb, :, pl.ds(obase + qq * NC, NC)]
                + xrecv[c].astype(jnp.float32))

        f_rdmas = []
        for j in range(2 * B):
            b, q2 = j // 2, j % 2
            o_row = pl.ds(my_s0, S_half)
            if q2 == 0:
                o_bf[b, o_row, :] = o_ref[b, o_row, :].astype(jnp.bfloat16)
            out_ref[b, :, pl.ds(nbase + q2 * 2 * NC, 2 * NC)] = jnp.dot(
                o_bf[b, o_row, :],
                w_bf[:, pl.ds(nbase + q2 * 2 * NC, 2 * NC)],
                preferred_element_type=jnp.float32)
            out_ref[b, :, pl.ds(obase + q2 * 2 * NC, 2 * NC)] = jnp.dot(
                o_bf[b, o_row, :],
                w_bf[:, pl.ds(obase + q2 * 2 * NC, 2 * NC)],
                preferred_element_type=jnp.float32)
            for t in range(2):
                q = q2 * 2 + t
                c = b * CPB + q
                y_rdmas[c].wait_recv()
                f = pltpu.make_async_remote_copy(
                    src_ref=yrecv.at[c], dst_ref=xrecv.at[c],
                    send_sem=fsend_sems.at[c], recv_sem=xrecv_sems.at[c],
                    device_id=xpeer, device_id_type=pl.DeviceIdType.MESH)
                f.start()
                f_rdmas.append(f)
                out_ref[b, :, pl.ds(nbase + q * NC, NC)] = (
                    out_ref[b, :, pl.ds(nbase + q * NC, NC)]
                    + yrecv[c].astype(jnp.float32))
                if c - LAG >= 0:
                    add_x_chunk(c - LAG)

        for c in range(C - LAG, C):
            add_x_chunk(c)

        for r in y_rdmas:
            r.wait_send()
        for r in f_rdmas:
            r.wait_send()

    return pl.pallas_call(
        body,
        out_shape=jax.ShapeDtypeStruct((B, S_half, N), jnp.float32),
        in_specs=[
            pl.BlockSpec(memory_space=pltpu.VMEM),
            pl.BlockSpec(memory_space=pltpu.VMEM),
        ],
        out_specs=pl.BlockSpec(memory_space=pltpu.VMEM),
        scratch_shapes=[
            pltpu.VMEM((B, S_full, K), jnp.bfloat16),
            pltpu.VMEM((K, N), jnp.bfloat16),
            pltpu.VMEM((C, S_half, NC), jnp.bfloat16),
            pltpu.VMEM((C, S_half, NC), jnp.bfloat16),
            pltpu.VMEM((C, S_half, NC), jnp.bfloat16),
            pltpu.SemaphoreType.DMA((C,)),
            pltpu.SemaphoreType.DMA((C,)),
            pltpu.SemaphoreType.DMA((C,)),
            pltpu.SemaphoreType.DMA((C,)),
        ],
        compiler_params=pltpu.CompilerParams(collective_id=0),
    )(O2, Wo)
